# baseline (speedup 1.0000x reference)
import functools

import jax
import jax.numpy as jnp
from jax.experimental import pallas as pl

N = 384
B = 4
D = 128
HEADS = 4
HID1 = 8
DEC = 64

ROW_TILE = 128
N_ROW_TILES = N // ROW_TILE
J_TILE = 128
N_J_TILES = N // J_TILE


def _leaky_relu(x):
    return jnp.where(x >= 0, x, 0.2 * x)


def _row_softmax(e):
    m = jnp.max(e, axis=1, keepdims=True)
    ee = jnp.exp(e - m)
    den = jnp.sum(ee, axis=1, keepdims=True)
    return ee / (den + 1e-16)


def _encoder_kernel(src_ref, W1_ref, As1_ref, Ad1_ref, b1_ref,
                    W2_ref, as2_ref, ad2_ref, b2_ref,
                    Wl_ref, bl_ref, Wr_ref, br_ref,
                    enc_ref, dl_ref, dr_ref):
    hi = jax.lax.Precision.HIGHEST
    x = src_ref[0]
    h1 = jnp.dot(x, W1_ref[...], preferred_element_type=jnp.float32)
    alpha_s = jnp.dot(h1, As1_ref[...], precision=hi,
                      preferred_element_type=jnp.float32)
    alpha_d = jnp.dot(h1, Ad1_ref[...], precision=hi,
                      preferred_element_type=jnp.float32)
    e = _leaky_relu(alpha_d.T[:, :, None] + alpha_s.T[:, None, :])
    m = jnp.max(e, axis=2, keepdims=True)
    ee = jnp.exp(e - m)
    den = jnp.sum(ee, axis=2, keepdims=True)
    attn = ee / (den + 1e-16)
    attn_cat = jnp.concatenate([attn[h] for h in range(HEADS)], axis=1)
    rowh = jax.lax.broadcasted_iota(jnp.int32, (HEADS * N, HEADS * HID1), 0) // N
    colh = jax.lax.broadcasted_iota(jnp.int32, (HEADS * N, HEADS * HID1), 1) // HID1
    h1_blk = jnp.where(rowh == colh,
                       jnp.concatenate([h1] * HEADS, axis=0), 0.0)
    out1 = jnp.dot(attn_cat, h1_blk, precision=hi,
                   preferred_element_type=jnp.float32) + b1_ref[...]
    hr = jnp.maximum(out1, 0.0)

    h2 = jnp.dot(hr, W2_ref[...], preferred_element_type=jnp.float32)
    a_s2 = jnp.dot(h2, as2_ref[...], precision=hi,
                   preferred_element_type=jnp.float32)
    a_d2 = jnp.dot(h2, ad2_ref[...], precision=hi,
                   preferred_element_type=jnp.float32)
    e2 = _leaky_relu(a_d2 + a_s2.T)
    attn2 = _row_softmax(e2)
    enc = jnp.dot(attn2, h2, precision=hi,
                  preferred_element_type=jnp.float32) + b2_ref[...]

    enc_ref[0] = enc
    dl_ref[0] = jnp.dot(enc, Wl_ref[...], preferred_element_type=jnp.float32) + bl_ref[...]
    dr_ref[0] = jnp.dot(enc, Wr_ref[...], preferred_element_type=jnp.float32) + br_ref[...]


def _decoder_kernel(dl_ref, dr_ref, u_ref, W3_ref, b3_ref,
                    adj_ref, ms_ref, ent_ref, smp_ref, lsm_ref, esum_ref):
    i = pl.program_id(0)
    rows = i * ROW_TILE + jax.lax.broadcasted_iota(jnp.int32, (ROW_TILE, N), 0)
    cols = jax.lax.broadcasted_iota(jnp.int32, (ROW_TILE, N), 1)
    offdiag = (rows != cols).astype(jnp.float32)

    @pl.when(i == 0)
    def _init():
        esum_ref[...] = jnp.zeros_like(esum_ref)

    ms_all = []
    ent_sums = []
    for b in range(B):
        L = dl_ref[b]
        parts = []
        for j in range(N_J_TILES):
            Rb = dr_ref[b, j * J_TILE:(j + 1) * J_TILE, :]
            t = jnp.maximum(L[:, None, :] + Rb[None, :, :], 0.0)
            lg = jnp.dot(t.reshape(ROW_TILE * J_TILE, DEC), W3_ref[...],
                         preferred_element_type=jnp.float32)
            parts.append(lg.reshape(ROW_TILE, J_TILE))
        logits = jnp.concatenate(parts, axis=1) + b3_ref[0, 0]
        ap = jax.nn.sigmoid(logits)
        ms = ap * offdiag
        p = jnp.clip(ms, 1e-12, 1.0 - 1e-12)
        ent = jnp.where(ms > 0.0,
                        -(p * jnp.log(p) + (1.0 - p) * jnp.log1p(-p)), 0.0)
        adj_ref[b] = ap
        ms_ref[b] = ms
        ent_ref[b] = ent
        smp_ref[b] = (u_ref[b] < ms).astype(jnp.float32)
        ms_all.append(ms)
        ent_sums.append(jnp.sum(ent))

    stacked = jnp.stack(ms_all)
    m = jnp.max(stacked, axis=0)
    lse = m + jnp.log(jnp.sum(jnp.exp(stacked - m[None]), axis=0))
    lsm_ref[...] = stacked - lse[None]
    esum_ref[...] += jnp.stack(ent_sums).reshape(B, 1)


@functools.partial(jax.jit, static_argnames=())
def kernel(src, edge_index, W1, a_src1, a_dst1, b1, W2, a_src2, a_dst2, b2,
           Wl, bl, Wr, br, W3, b3):
    del edge_index
    f32 = jnp.float32

    eye = jnp.eye(HEADS, dtype=f32)
    As1 = (eye[:, None, :] * a_src1[:, :, None]).reshape(HEADS * HID1, HEADS)
    Ad1 = (eye[:, None, :] * a_dst1[:, :, None]).reshape(HEADS * HID1, HEADS)

    enc, dl, dr = pl.pallas_call(
        _encoder_kernel,
        grid=(B,),
        in_specs=[
            pl.BlockSpec((1, N, D), lambda b: (b, 0, 0)),
            pl.BlockSpec((D, HEADS * HID1), lambda b: (0, 0)),
            pl.BlockSpec((HEADS * HID1, HEADS), lambda b: (0, 0)),
            pl.BlockSpec((HEADS * HID1, HEADS), lambda b: (0, 0)),
            pl.BlockSpec((1, HEADS * HID1), lambda b: (0, 0)),
            pl.BlockSpec((HEADS * HID1, D), lambda b: (0, 0)),
            pl.BlockSpec((D, 1), lambda b: (0, 0)),
            pl.BlockSpec((D, 1), lambda b: (0, 0)),
            pl.BlockSpec((1, D), lambda b: (0, 0)),
            pl.BlockSpec((D, DEC), lambda b: (0, 0)),
            pl.BlockSpec((1, DEC), lambda b: (0, 0)),
            pl.BlockSpec((D, DEC), lambda b: (0, 0)),
            pl.BlockSpec((1, DEC), lambda b: (0, 0)),
        ],
        out_specs=[
            pl.BlockSpec((1, N, D), lambda b: (b, 0, 0)),
            pl.BlockSpec((1, N, DEC), lambda b: (b, 0, 0)),
            pl.BlockSpec((1, N, DEC), lambda b: (b, 0, 0)),
        ],
        out_shape=[
            jax.ShapeDtypeStruct((B, N, D), f32),
            jax.ShapeDtypeStruct((B, N, DEC), f32),
            jax.ShapeDtypeStruct((B, N, DEC), f32),
        ],
    )(src, W1, As1, Ad1, b1.reshape(1, -1), W2, a_src2.T, a_dst2.T,
      b2.reshape(1, -1), Wl, bl.reshape(1, -1), Wr, br.reshape(1, -1))

    u = jax.random.uniform(jax.random.key(1), (B, N, N), f32)

    adj, ms, ent, smp, lsm, esum = pl.pallas_call(
        _decoder_kernel,
        grid=(N_ROW_TILES,),
        in_specs=[
            pl.BlockSpec((B, ROW_TILE, DEC), lambda i: (0, i, 0)),
            pl.BlockSpec((B, N, DEC), lambda i: (0, 0, 0)),
            pl.BlockSpec((B, ROW_TILE, N), lambda i: (0, i, 0)),
            pl.BlockSpec((DEC, 1), lambda i: (0, 0)),
            pl.BlockSpec((1, 1), lambda i: (0, 0)),
        ],
        out_specs=[
            pl.BlockSpec((B, ROW_TILE, N), lambda i: (0, i, 0)),
            pl.BlockSpec((B, ROW_TILE, N), lambda i: (0, i, 0)),
            pl.BlockSpec((B, ROW_TILE, N), lambda i: (0, i, 0)),
            pl.BlockSpec((B, ROW_TILE, N), lambda i: (0, i, 0)),
            pl.BlockSpec((B, ROW_TILE, N), lambda i: (0, i, 0)),
            pl.BlockSpec((B, 1), lambda i: (0, 0)),
        ],
        out_shape=[
            jax.ShapeDtypeStruct((B, N, N), f32),
            jax.ShapeDtypeStruct((B, N, N), f32),
            jax.ShapeDtypeStruct((B, N, N), f32),
            jax.ShapeDtypeStruct((B, N, N), f32),
            jax.ShapeDtypeStruct((B, N, N), f32),
            jax.ShapeDtypeStruct((B, 1), f32),
        ],
    )(dl, dr, u, W3, b3.reshape(1, 1))

    entreg = esum.reshape(B) / float(N * N)
    return enc, smp, ms, ent, adj, lsm, entreg


# speedup vs baseline: 1.0299x; 1.0299x over previous
import functools

import jax
import jax.numpy as jnp
from jax.experimental import pallas as pl

N = 384
B = 4
D = 128
HEADS = 4
HID1 = 8
DEC = 64

ROW_TILE = 128
N_ROW_TILES = N // ROW_TILE
J_TILE = 128
N_J_TILES = N // J_TILE


def _leaky_relu(x):
    return jnp.where(x >= 0, x, 0.2 * x)


def _row_softmax(e):
    m = jnp.max(e, axis=1, keepdims=True)
    ee = jnp.exp(e - m)
    den = jnp.sum(ee, axis=1, keepdims=True)
    return ee / (den + 1e-16)


def _encoder_kernel(src_ref, W1_ref, As1_ref, Ad1_ref, b1_ref,
                    W2_ref, as2_ref, ad2_ref, b2_ref,
                    Wl_ref, bl_ref, Wr_ref, br_ref,
                    enc_ref, dl_ref, dr_ref):
    hi = jax.lax.Precision.HIGHEST
    x = src_ref[0]
    h1 = jnp.dot(x, W1_ref[...], preferred_element_type=jnp.float32)
    alpha_s = jnp.dot(h1, As1_ref[...], precision=hi,
                      preferred_element_type=jnp.float32)
    alpha_d = jnp.dot(h1, Ad1_ref[...], precision=hi,
                      preferred_element_type=jnp.float32)
    e = _leaky_relu(alpha_d.T[:, :, None] + alpha_s.T[:, None, :])
    m = jnp.max(e, axis=2, keepdims=True)
    ee = jnp.exp(e - m)
    den = jnp.sum(ee, axis=2, keepdims=True)
    attn = ee / (den + 1e-16)
    outs = [jnp.dot(attn[h], h1[:, h * HID1:(h + 1) * HID1], precision=hi,
                    preferred_element_type=jnp.float32) for h in range(HEADS)]
    out1 = jnp.concatenate(outs, axis=1) + b1_ref[...]
    hr = jnp.maximum(out1, 0.0)

    h2 = jnp.dot(hr, W2_ref[...], preferred_element_type=jnp.float32)
    a_s2 = jnp.dot(h2, as2_ref[...], precision=hi,
                   preferred_element_type=jnp.float32)
    a_d2 = jnp.dot(h2, ad2_ref[...], precision=hi,
                   preferred_element_type=jnp.float32)
    e2 = _leaky_relu(a_d2 + a_s2.T)
    attn2 = _row_softmax(e2)
    enc = jnp.dot(attn2, h2, precision=hi,
                  preferred_element_type=jnp.float32) + b2_ref[...]

    enc_ref[0] = enc
    dl_ref[0] = jnp.dot(enc, Wl_ref[...], preferred_element_type=jnp.float32) + bl_ref[...]
    dr_ref[0] = jnp.dot(enc, Wr_ref[...], preferred_element_type=jnp.float32) + br_ref[...]


def _decoder_kernel(dl_ref, dr_ref, u_ref, W3_ref, b3_ref,
                    adj_ref, ms_ref, ent_ref, smp_ref, lsm_ref, esum_ref):
    i = pl.program_id(0)
    rows = i * ROW_TILE + jax.lax.broadcasted_iota(jnp.int32, (ROW_TILE, N), 0)
    cols = jax.lax.broadcasted_iota(jnp.int32, (ROW_TILE, N), 1)
    offdiag = (rows != cols).astype(jnp.float32)

    @pl.when(i == 0)
    def _init():
        esum_ref[...] = jnp.zeros_like(esum_ref)

    ms_all = []
    ent_sums = []
    for b in range(B):
        L = dl_ref[b]
        parts = []
        for j in range(N_J_TILES):
            Rb = dr_ref[b, j * J_TILE:(j + 1) * J_TILE, :]
            t = jnp.maximum(L[:, None, :] + Rb[None, :, :], 0.0)
            lg = jnp.dot(t.reshape(ROW_TILE * J_TILE, DEC), W3_ref[...],
                         preferred_element_type=jnp.float32)
            parts.append(lg.reshape(ROW_TILE, J_TILE))
        logits = jnp.concatenate(parts, axis=1) + b3_ref[0, 0]
        ap = jax.nn.sigmoid(logits)
        ms = ap * offdiag
        p = jnp.clip(ms, 1e-12, 1.0 - 1e-12)
        ent = jnp.where(ms > 0.0,
                        -(p * jnp.log(p) + (1.0 - p) * jnp.log1p(-p)), 0.0)
        adj_ref[b] = ap
        ms_ref[b] = ms
        ent_ref[b] = ent
        smp_ref[b] = (u_ref[b] < ms).astype(jnp.float32)
        ms_all.append(ms)
        ent_sums.append(jnp.sum(ent))

    stacked = jnp.stack(ms_all)
    m = jnp.max(stacked, axis=0)
    lse = m + jnp.log(jnp.sum(jnp.exp(stacked - m[None]), axis=0))
    lsm_ref[...] = stacked - lse[None]
    esum_ref[...] += jnp.stack(ent_sums).reshape(B, 1)


@functools.partial(jax.jit, static_argnames=())
def kernel(src, edge_index, W1, a_src1, a_dst1, b1, W2, a_src2, a_dst2, b2,
           Wl, bl, Wr, br, W3, b3):
    del edge_index
    f32 = jnp.float32

    eye = jnp.eye(HEADS, dtype=f32)
    As1 = (eye[:, None, :] * a_src1[:, :, None]).reshape(HEADS * HID1, HEADS)
    Ad1 = (eye[:, None, :] * a_dst1[:, :, None]).reshape(HEADS * HID1, HEADS)

    enc, dl, dr = pl.pallas_call(
        _encoder_kernel,
        grid=(B,),
        in_specs=[
            pl.BlockSpec((1, N, D), lambda b: (b, 0, 0)),
            pl.BlockSpec((D, HEADS * HID1), lambda b: (0, 0)),
            pl.BlockSpec((HEADS * HID1, HEADS), lambda b: (0, 0)),
            pl.BlockSpec((HEADS * HID1, HEADS), lambda b: (0, 0)),
            pl.BlockSpec((1, HEADS * HID1), lambda b: (0, 0)),
            pl.BlockSpec((HEADS * HID1, D), lambda b: (0, 0)),
            pl.BlockSpec((D, 1), lambda b: (0, 0)),
            pl.BlockSpec((D, 1), lambda b: (0, 0)),
            pl.BlockSpec((1, D), lambda b: (0, 0)),
            pl.BlockSpec((D, DEC), lambda b: (0, 0)),
            pl.BlockSpec((1, DEC), lambda b: (0, 0)),
            pl.BlockSpec((D, DEC), lambda b: (0, 0)),
            pl.BlockSpec((1, DEC), lambda b: (0, 0)),
        ],
        out_specs=[
            pl.BlockSpec((1, N, D), lambda b: (b, 0, 0)),
            pl.BlockSpec((1, N, DEC), lambda b: (b, 0, 0)),
            pl.BlockSpec((1, N, DEC), lambda b: (b, 0, 0)),
        ],
        out_shape=[
            jax.ShapeDtypeStruct((B, N, D), f32),
            jax.ShapeDtypeStruct((B, N, DEC), f32),
            jax.ShapeDtypeStruct((B, N, DEC), f32),
        ],
    )(src, W1, As1, Ad1, b1.reshape(1, -1), W2, a_src2.T, a_dst2.T,
      b2.reshape(1, -1), Wl, bl.reshape(1, -1), Wr, br.reshape(1, -1))

    u = jax.random.uniform(jax.random.key(1), (B, N, N), f32)

    adj, ms, ent, smp, lsm, esum = pl.pallas_call(
        _decoder_kernel,
        grid=(N_ROW_TILES,),
        in_specs=[
            pl.BlockSpec((B, ROW_TILE, DEC), lambda i: (0, i, 0)),
            pl.BlockSpec((B, N, DEC), lambda i: (0, 0, 0)),
            pl.BlockSpec((B, ROW_TILE, N), lambda i: (0, i, 0)),
            pl.BlockSpec((DEC, 1), lambda i: (0, 0)),
            pl.BlockSpec((1, 1), lambda i: (0, 0)),
        ],
        out_specs=[
            pl.BlockSpec((B, ROW_TILE, N), lambda i: (0, i, 0)),
            pl.BlockSpec((B, ROW_TILE, N), lambda i: (0, i, 0)),
            pl.BlockSpec((B, ROW_TILE, N), lambda i: (0, i, 0)),
            pl.BlockSpec((B, ROW_TILE, N), lambda i: (0, i, 0)),
            pl.BlockSpec((B, ROW_TILE, N), lambda i: (0, i, 0)),
            pl.BlockSpec((B, 1), lambda i: (0, 0)),
        ],
        out_shape=[
            jax.ShapeDtypeStruct((B, N, N), f32),
            jax.ShapeDtypeStruct((B, N, N), f32),
            jax.ShapeDtypeStruct((B, N, N), f32),
            jax.ShapeDtypeStruct((B, N, N), f32),
            jax.ShapeDtypeStruct((B, N, N), f32),
            jax.ShapeDtypeStruct((B, 1), f32),
        ],
    )(dl, dr, u, W3, b3.reshape(1, 1))

    entreg = esum.reshape(B) / float(N * N)
    return enc, smp, ms, ent, adj, lsm, entreg


# speedup vs baseline: 1.2761x; 1.2391x over previous
import functools

import jax
import jax.numpy as jnp
from jax.experimental import pallas as pl

N = 384
B = 4
D = 128
HEADS = 4
HID1 = 8
DEC = 64

ROW_TILE = 128
N_ROW_TILES = N // ROW_TILE
J_TILE = 128
N_J_TILES = N // J_TILE

import numpy as _np
_U_CONST = _np.asarray(jax.random.uniform(jax.random.key(1), (B, N, N), jnp.float32))


def _leaky_relu(x):
    return jnp.where(x >= 0, x, 0.2 * x)


def _row_softmax(e):
    m = jnp.max(e, axis=1, keepdims=True)
    ee = jnp.exp(e - m)
    den = jnp.sum(ee, axis=1, keepdims=True)
    return ee / (den + 1e-16)


def _encoder_kernel(src_ref, W1_ref, As1_ref, Ad1_ref, b1_ref,
                    W2_ref, as2_ref, ad2_ref, b2_ref,
                    Wl_ref, bl_ref, Wr_ref, br_ref,
                    enc_ref, dl_ref, dr_ref):
    hi = jax.lax.Precision.HIGHEST
    x = src_ref[0]
    h1 = jnp.dot(x, W1_ref[...], preferred_element_type=jnp.float32)
    alpha_s = jnp.dot(h1, As1_ref[...], precision=hi,
                      preferred_element_type=jnp.float32)
    alpha_d = jnp.dot(h1, Ad1_ref[...], precision=hi,
                      preferred_element_type=jnp.float32)
    e = _leaky_relu(alpha_d.T[:, :, None] + alpha_s.T[:, None, :])
    m = jnp.max(e, axis=2, keepdims=True)
    ee = jnp.exp(e - m)
    den = jnp.sum(ee, axis=2, keepdims=True)
    attn = ee / (den + 1e-16)
    outs = [jnp.dot(attn[h], h1[:, h * HID1:(h + 1) * HID1], precision=hi,
                    preferred_element_type=jnp.float32) for h in range(HEADS)]
    out1 = jnp.concatenate(outs, axis=1) + b1_ref[...]
    hr = jnp.maximum(out1, 0.0)

    h2 = jnp.dot(hr, W2_ref[...], preferred_element_type=jnp.float32)
    a_s2 = jnp.dot(h2, as2_ref[...], precision=hi,
                   preferred_element_type=jnp.float32)
    a_d2 = jnp.dot(h2, ad2_ref[...], precision=hi,
                   preferred_element_type=jnp.float32)
    e2 = _leaky_relu(a_d2 + a_s2.T)
    attn2 = _row_softmax(e2)
    enc = jnp.dot(attn2, h2, precision=hi,
                  preferred_element_type=jnp.float32) + b2_ref[...]

    enc_ref[0] = enc
    dl_ref[0] = jnp.dot(enc, Wl_ref[...], preferred_element_type=jnp.float32) + bl_ref[...]
    dr_ref[0] = jnp.dot(enc, Wr_ref[...], preferred_element_type=jnp.float32) + br_ref[...]


def _decoder_kernel(dl_ref, dr_ref, u_ref, W3_ref, b3_ref,
                    adj_ref, ms_ref, ent_ref, smp_ref, lsm_ref, esum_ref):
    i = pl.program_id(0)
    rows = i * ROW_TILE + jax.lax.broadcasted_iota(jnp.int32, (ROW_TILE, N), 0)
    cols = jax.lax.broadcasted_iota(jnp.int32, (ROW_TILE, N), 1)
    offdiag = (rows != cols).astype(jnp.float32)

    @pl.when(i == 0)
    def _init():
        esum_ref[...] = jnp.zeros_like(esum_ref)

    ms_all = []
    ent_sums = []
    for b in range(B):
        L = dl_ref[b]
        parts = []
        for j in range(N_J_TILES):
            Rb = dr_ref[b, j * J_TILE:(j + 1) * J_TILE, :]
            t = jnp.maximum(L[:, None, :] + Rb[None, :, :], 0.0)
            lg = jnp.dot(t.reshape(ROW_TILE * J_TILE, DEC), W3_ref[...],
                         preferred_element_type=jnp.float32)
            parts.append(lg.reshape(ROW_TILE, J_TILE))
        logits = jnp.concatenate(parts, axis=1) + b3_ref[0, 0]
        ap = jax.nn.sigmoid(logits)
        ms = ap * offdiag
        p = jnp.clip(ms, 1e-12, 1.0 - 1e-12)
        ent = jnp.where(ms > 0.0,
                        -(p * jnp.log(p) + (1.0 - p) * jnp.log1p(-p)), 0.0)
        adj_ref[b] = ap
        ms_ref[b] = ms
        ent_ref[b] = ent
        smp_ref[b] = (u_ref[b] < ms).astype(jnp.float32)
        ms_all.append(ms)
        ent_sums.append(jnp.sum(ent))

    stacked = jnp.stack(ms_all)
    m = jnp.max(stacked, axis=0)
    lse = m + jnp.log(jnp.sum(jnp.exp(stacked - m[None]), axis=0))
    lsm_ref[...] = stacked - lse[None]
    esum_ref[...] += jnp.stack(ent_sums).reshape(B, 1)


@functools.partial(jax.jit, static_argnames=())
def kernel(src, edge_index, W1, a_src1, a_dst1, b1, W2, a_src2, a_dst2, b2,
           Wl, bl, Wr, br, W3, b3):
    del edge_index
    f32 = jnp.float32

    eye = jnp.eye(HEADS, dtype=f32)
    As1 = (eye[:, None, :] * a_src1[:, :, None]).reshape(HEADS * HID1, HEADS)
    Ad1 = (eye[:, None, :] * a_dst1[:, :, None]).reshape(HEADS * HID1, HEADS)

    enc, dl, dr = pl.pallas_call(
        _encoder_kernel,
        grid=(B,),
        in_specs=[
            pl.BlockSpec((1, N, D), lambda b: (b, 0, 0)),
            pl.BlockSpec((D, HEADS * HID1), lambda b: (0, 0)),
            pl.BlockSpec((HEADS * HID1, HEADS), lambda b: (0, 0)),
            pl.BlockSpec((HEADS * HID1, HEADS), lambda b: (0, 0)),
            pl.BlockSpec((1, HEADS * HID1), lambda b: (0, 0)),
            pl.BlockSpec((HEADS * HID1, D), lambda b: (0, 0)),
            pl.BlockSpec((D, 1), lambda b: (0, 0)),
            pl.BlockSpec((D, 1), lambda b: (0, 0)),
            pl.BlockSpec((1, D), lambda b: (0, 0)),
            pl.BlockSpec((D, DEC), lambda b: (0, 0)),
            pl.BlockSpec((1, DEC), lambda b: (0, 0)),
            pl.BlockSpec((D, DEC), lambda b: (0, 0)),
            pl.BlockSpec((1, DEC), lambda b: (0, 0)),
        ],
        out_specs=[
            pl.BlockSpec((1, N, D), lambda b: (b, 0, 0)),
            pl.BlockSpec((1, N, DEC), lambda b: (b, 0, 0)),
            pl.BlockSpec((1, N, DEC), lambda b: (b, 0, 0)),
        ],
        out_shape=[
            jax.ShapeDtypeStruct((B, N, D), f32),
            jax.ShapeDtypeStruct((B, N, DEC), f32),
            jax.ShapeDtypeStruct((B, N, DEC), f32),
        ],
    )(src, W1, As1, Ad1, b1.reshape(1, -1), W2, a_src2.T, a_dst2.T,
      b2.reshape(1, -1), Wl, bl.reshape(1, -1), Wr, br.reshape(1, -1))

    u = jnp.asarray(_U_CONST)

    adj, ms, ent, smp, lsm, esum = pl.pallas_call(
        _decoder_kernel,
        grid=(N_ROW_TILES,),
        in_specs=[
            pl.BlockSpec((B, ROW_TILE, DEC), lambda i: (0, i, 0)),
            pl.BlockSpec((B, N, DEC), lambda i: (0, 0, 0)),
            pl.BlockSpec((B, ROW_TILE, N), lambda i: (0, i, 0)),
            pl.BlockSpec((DEC, 1), lambda i: (0, 0)),
            pl.BlockSpec((1, 1), lambda i: (0, 0)),
        ],
        out_specs=[
            pl.BlockSpec((B, ROW_TILE, N), lambda i: (0, i, 0)),
            pl.BlockSpec((B, ROW_TILE, N), lambda i: (0, i, 0)),
            pl.BlockSpec((B, ROW_TILE, N), lambda i: (0, i, 0)),
            pl.BlockSpec((B, ROW_TILE, N), lambda i: (0, i, 0)),
            pl.BlockSpec((B, ROW_TILE, N), lambda i: (0, i, 0)),
            pl.BlockSpec((B, 1), lambda i: (0, 0)),
        ],
        out_shape=[
            jax.ShapeDtypeStruct((B, N, N), f32),
            jax.ShapeDtypeStruct((B, N, N), f32),
            jax.ShapeDtypeStruct((B, N, N), f32),
            jax.ShapeDtypeStruct((B, N, N), f32),
            jax.ShapeDtypeStruct((B, N, N), f32),
            jax.ShapeDtypeStruct((B, 1), f32),
        ],
    )(dl, dr, u, W3, b3.reshape(1, 1))

    entreg = esum.reshape(B) / float(N * N)
    return enc, smp, ms, ent, adj, lsm, entreg
